# baseline (speedup 1.0000x reference)
import functools

import jax
import jax.numpy as jnp
import numpy as np
from jax.experimental import pallas as pl


_FACTOR = 5


def _m_topk_kernel(q_ref, k_ref, c_ref, mtop_ref, *, L, S, D, U, KT):
    q = q_ref[...].astype(jnp.bfloat16)
    nk = S // KT
    max_acc = None
    sum_acc = None
    for t in range(nk):
        kt = k_ref[t * KT:(t + 1) * KT, :].astype(jnp.bfloat16)
        ct = c_ref[:, t * KT:(t + 1) * KT]
        s = jax.lax.dot_general(
            q, kt, (((1,), (1,)), ((), ())),
            preferred_element_type=jnp.float32)
        masked = jnp.where(ct > 0.0, s, -3e38)
        tmax = jnp.max(masked, axis=1, keepdims=True)
        tsum = jnp.sum(s * ct, axis=1, keepdims=True)
        if t == 0:
            max_acc, sum_acc = tmax, tsum
        else:
            max_acc = jnp.maximum(max_acc, tmax)
            sum_acc = sum_acc + tsum
    m = max_acc - sum_acc * (1.0 / S)

    rows = L // 128
    mr = m.reshape(rows, 128)
    sub = jax.lax.broadcasted_iota(jnp.int32, (rows, 128), 0)
    lane = jax.lax.broadcasted_iota(jnp.int32, (rows, 128), 1)
    flat = sub * 128 + lane
    lane_v = jax.lax.broadcasted_iota(jnp.int32, (1, 128), 1)
    idxv = jnp.full((1, 128), S + 1000, dtype=jnp.int32)

    def body(i, carry):
        mr, idxv = carry
        cm = jnp.max(mr)
        cand = jnp.where(mr == cm, flat, jnp.int32(2147480000))
        pos = jnp.min(cand)
        mr = jnp.where(flat == pos, -3e38, mr)
        idxv = jnp.where(lane_v == i, pos, idxv)
        return mr, idxv

    _, idxv = jax.lax.fori_loop(0, U, body, (mr, idxv))
    mtop_ref[...] = idxv.reshape(1, 1, 128)


def _attn_ctx_kernel(q_ref, k_ref, v_ref, mtop_ref, out_ref, *, L, S, D, U,
                     scale):
    idxs = mtop_ref[0]
    idx64 = idxs[:, :64]
    idx_col = jnp.swapaxes(idx64, 0, 1)

    col = jax.lax.broadcasted_iota(jnp.int32, (64, S), 1)
    oh = (idx_col == col).astype(jnp.float32)
    oh_b = oh.astype(jnp.bfloat16)

    q = q_ref[...]
    k = k_ref[...]
    v = v_ref[...]

    qr = jax.lax.dot_general(
        oh_b, q.astype(jnp.bfloat16), (((1,), (0,)), ((), ())),
        preferred_element_type=jnp.float32)

    sc = jax.lax.dot_general(
        qr.astype(jnp.bfloat16), k.astype(jnp.bfloat16),
        (((1,), (1,)), ((), ())),
        preferred_element_type=jnp.float32) * scale

    sc = jnp.where(col > idx_col, -1e9, sc)
    sc = sc - jnp.max(sc, axis=1, keepdims=True)
    e = jnp.exp(sc)
    attn = e / jnp.sum(e, axis=1, keepdims=True)

    att = jax.lax.dot_general(
        attn.astype(jnp.bfloat16), v.astype(jnp.bfloat16),
        (((1,), (0,)), ((), ())),
        preferred_element_type=jnp.float32)

    ctx = v
    sh = 1
    while sh < L:
        ctx = ctx + jnp.concatenate(
            [jnp.zeros((sh, D), jnp.float32), ctx[:L - sh, :]], axis=0)
        sh *= 2

    contrib = jax.lax.dot_general(
        oh, att, (((0,), (0,)), ((), ())),
        preferred_element_type=jnp.float32,
        precision=jax.lax.Precision.HIGHEST)
    sel = jax.lax.dot_general(
        oh, jnp.ones((64, D), jnp.float32), (((0,), (0,)), ((), ())),
        preferred_element_type=jnp.float32)
    out_ref[...] = jnp.where(sel > 0.0, contrib, ctx)


def _build_cmat(L, S, u):
    idx_key = jax.random.key(42)
    index_sample = jax.random.randint(idx_key, (L, u), 0, S)
    return jnp.zeros((L, S), jnp.float32).at[
        jnp.arange(L)[:, None], index_sample].add(1.0)


@functools.lru_cache(maxsize=None)
def _cmat_eager(L, S, u):
    with jax.ensure_compile_time_eval():
        return jax.block_until_ready(_build_cmat(L, S, u))


def _sample_cmat(L, S, u):
    try:
        return _cmat_eager(L, S, u)
    except Exception:
        return _build_cmat(L, S, u)


def kernel(queries, keys, values):
    B, L, H, D = queries.shape
    S = keys.shape[1]
    u = min(max(1, int(_FACTOR * np.log(max(L, 2)))), L)
    return _run(queries, keys, values, _sample_cmat(L, S, u))


@jax.jit
def _run(queries, keys, values, cmat):
    B, L, H, D = queries.shape
    S = keys.shape[1]
    U = min(max(1, int(_FACTOR * np.log(max(S, 2)))), S)
    scale = 1.0 / np.sqrt(D)

    Qs = queries.reshape(L, H * D)
    Ks = keys.reshape(L, H * D)
    Vs = values.reshape(L, H * D)

    KT = 512
    m_topk = pl.pallas_call(
        functools.partial(_m_topk_kernel, L=L, S=S, D=D, U=U, KT=KT),
        grid=(H,),
        in_specs=[
            pl.BlockSpec((L, D), lambda h: (0, h)),
            pl.BlockSpec((S, D), lambda h: (0, h)),
            pl.BlockSpec((L, S), lambda h: (0, 0)),
        ],
        out_specs=pl.BlockSpec((1, 1, 128), lambda h: (h, 0, 0)),
        out_shape=jax.ShapeDtypeStruct((H, 1, 128), jnp.int32),
    )
    mtop = m_topk(Qs, Ks, cmat)

    attn_ctx = pl.pallas_call(
        functools.partial(_attn_ctx_kernel, L=L, S=S, D=D, U=U, scale=scale),
        grid=(H,),
        in_specs=[
            pl.BlockSpec((L, D), lambda h: (0, h)),
            pl.BlockSpec((S, D), lambda h: (0, h)),
            pl.BlockSpec((S, D), lambda h: (0, h)),
            pl.BlockSpec((1, 1, 128), lambda h: (h, 0, 0)),
        ],
        out_specs=pl.BlockSpec((S, D), lambda h: (0, h)),
        out_shape=jax.ShapeDtypeStruct((S, H * D), jnp.float32),
    )
    out = attn_ctx(Qs, Ks, Vs, mtop)
    return out.reshape(B, L, H, D)


# speedup vs baseline: 1.7998x; 1.7998x over previous
import functools

import jax
import jax.numpy as jnp
import numpy as np
from jax.experimental import pallas as pl
from jax.experimental.pallas import tpu as pltpu


_FACTOR = 5


def _m_kernel(q_ref, k_ref, c_ref, m_ref, *, L, S, D, KT):
    q = q_ref[...].astype(jnp.bfloat16)
    nk = S // KT
    max_acc = None
    sum_acc = None
    for t in range(nk):
        kt = k_ref[t * KT:(t + 1) * KT, :].astype(jnp.bfloat16)
        ct = c_ref[:, t * KT:(t + 1) * KT]
        s = jax.lax.dot_general(
            q, kt, (((1,), (1,)), ((), ())),
            preferred_element_type=jnp.float32)
        masked = jnp.where(ct > 0.0, s, -3e38)
        tmax = jnp.max(masked, axis=1, keepdims=True)
        tsum = jnp.sum(s * ct, axis=1, keepdims=True)
        if t == 0:
            max_acc, sum_acc = tmax, tsum
        else:
            max_acc = jnp.maximum(max_acc, tmax)
            sum_acc = sum_acc + tsum
    m = max_acc - sum_acc * (1.0 / S)
    m_ref[...] = m.reshape(1, 1, L)


def _topk_kernel(m_ref, mtop_ref, *, H, S, U):
    m = m_ref[...].reshape(H, S)
    col = jax.lax.broadcasted_iota(jnp.int32, (H, S), 1)
    lane = jax.lax.broadcasted_iota(jnp.int32, (H, 128), 1)
    idxv = jnp.full((H, 128), S + 1000, dtype=jnp.int32)

    def body(i, carry):
        m, idxv = carry
        cm = jnp.max(m, axis=1, keepdims=True)
        cand = jnp.where(m == cm, col, jnp.int32(2147480000))
        pos = jnp.min(cand, axis=1, keepdims=True)
        m = jnp.where(col == pos, -3e38, m)
        idxv = jnp.where(lane == i, pos, idxv)
        return m, idxv

    _, idxv = jax.lax.fori_loop(0, U, body, (m, idxv))
    mtop_ref[...] = idxv.reshape(H, 1, 128)


def _attn_ctx_kernel(q_ref, k_ref, v_ref, mtop_ref, idx_smem, out_ref, *,
                     L, S, D, U, scale):
    idxs = mtop_ref[0]
    idx64 = idxs[:, :64]
    idx_col = jnp.swapaxes(idx64, 0, 1)

    col = jax.lax.broadcasted_iota(jnp.int32, (64, S), 1)
    oh = (idx_col == col).astype(jnp.bfloat16)

    q = q_ref[...]
    k = k_ref[...]
    v = v_ref[...]

    qr = jax.lax.dot_general(
        oh, q.astype(jnp.bfloat16), (((1,), (0,)), ((), ())),
        preferred_element_type=jnp.float32)

    sc = jax.lax.dot_general(
        qr.astype(jnp.bfloat16), k.astype(jnp.bfloat16),
        (((1,), (1,)), ((), ())),
        preferred_element_type=jnp.float32) * scale

    sc = jnp.where(col > idx_col, -1e9, sc)
    sc = sc - jnp.max(sc, axis=1, keepdims=True)
    e = jnp.exp(sc)
    attn = e / jnp.sum(e, axis=1, keepdims=True)

    att = jax.lax.dot_general(
        attn.astype(jnp.bfloat16), v.astype(jnp.bfloat16),
        (((1,), (0,)), ((), ())),
        preferred_element_type=jnp.float32)

    ctx = v
    sh = 1
    while sh < L:
        ctx = ctx + jnp.concatenate(
            [jnp.zeros((sh, D), jnp.float32), ctx[:L - sh, :]], axis=0)
        sh *= 2
    out_ref[...] = ctx

    for i in range(U):
        row = idx_smem[0, 0, i]
        out_ref[pl.ds(row, 1), :] = att[i:i + 1, :]


def _build_cmat(L, S, u):
    idx_key = jax.random.key(42)
    index_sample = jax.random.randint(idx_key, (L, u), 0, S)
    return jnp.zeros((L, S), jnp.float32).at[
        jnp.arange(L)[:, None], index_sample].add(1.0)


@functools.lru_cache(maxsize=None)
def _cmat_eager(L, S, u):
    with jax.ensure_compile_time_eval():
        return jax.block_until_ready(_build_cmat(L, S, u))


def _sample_cmat(L, S, u):
    try:
        return _cmat_eager(L, S, u)
    except Exception:
        return _build_cmat(L, S, u)


def kernel(queries, keys, values):
    B, L, H, D = queries.shape
    S = keys.shape[1]
    u = min(max(1, int(_FACTOR * np.log(max(L, 2)))), L)
    return _run(queries, keys, values, _sample_cmat(L, S, u))


@jax.jit
def _run(queries, keys, values, cmat):
    B, L, H, D = queries.shape
    S = keys.shape[1]
    U = min(max(1, int(_FACTOR * np.log(max(S, 2)))), S)
    scale = 1.0 / np.sqrt(D)

    Qs = queries.reshape(L, H * D)
    Ks = keys.reshape(L, H * D)
    Vs = values.reshape(L, H * D)

    m_call = pl.pallas_call(
        functools.partial(_m_kernel, L=L, S=S, D=D, KT=512),
        grid=(H,),
        in_specs=[
            pl.BlockSpec((L, D), lambda h: (0, h)),
            pl.BlockSpec((S, D), lambda h: (0, h)),
            pl.BlockSpec((L, S), lambda h: (0, 0)),
        ],
        out_specs=pl.BlockSpec((1, 1, L), lambda h: (h, 0, 0)),
        out_shape=jax.ShapeDtypeStruct((H, 1, L), jnp.float32),
    )
    m = m_call(Qs, Ks, cmat)

    topk_call = pl.pallas_call(
        functools.partial(_topk_kernel, H=H, S=S, U=U),
        grid=(1,),
        in_specs=[pl.BlockSpec((H, 1, S), lambda i: (0, 0, 0))],
        out_specs=pl.BlockSpec((H, 1, 128), lambda i: (0, 0, 0)),
        out_shape=jax.ShapeDtypeStruct((H, 1, 128), jnp.int32),
    )
    mtop = topk_call(m)

    attn_ctx = pl.pallas_call(
        functools.partial(_attn_ctx_kernel, L=L, S=S, D=D, U=U, scale=scale),
        grid=(H,),
        in_specs=[
            pl.BlockSpec((L, D), lambda h: (0, h)),
            pl.BlockSpec((S, D), lambda h: (0, h)),
            pl.BlockSpec((S, D), lambda h: (0, h)),
            pl.BlockSpec((1, 1, 128), lambda h: (h, 0, 0)),
            pl.BlockSpec((1, 1, 128), lambda h: (h, 0, 0),
                         memory_space=pltpu.SMEM),
        ],
        out_specs=pl.BlockSpec((S, D), lambda h: (0, h)),
        out_shape=jax.ShapeDtypeStruct((S, H * D), jnp.float32),
    )
    out = attn_ctx(Qs, Ks, Vs, mtop, mtop)
    return out.reshape(B, L, H, D)


# speedup vs baseline: 1.8259x; 1.0145x over previous
import functools

import jax
import jax.numpy as jnp
import numpy as np
from jax.experimental import pallas as pl
from jax.experimental.pallas import tpu as pltpu


_FACTOR = 5


def _m_kernel(q_ref, k_ref, c_ref, g_ref, m_ref, *, L, S, D, KT):
    q = q_ref[...].astype(jnp.bfloat16)
    nk = S // KT
    max_acc = None
    sum_acc = None
    for t in range(nk):
        kt = k_ref[t * KT:(t + 1) * KT, :].astype(jnp.bfloat16)
        ct = c_ref[:, t * KT:(t + 1) * KT]
        gt = g_ref[:, t * KT:(t + 1) * KT]
        s = jax.lax.dot_general(
            q, kt, (((1,), (1,)), ((), ())),
            preferred_element_type=jnp.float32)
        masked = s + gt
        tmax = jnp.max(masked, axis=1, keepdims=True)
        tsum = jnp.sum(s * ct, axis=1, keepdims=True)
        if t == 0:
            max_acc, sum_acc = tmax, tsum
        else:
            max_acc = jnp.maximum(max_acc, tmax)
            sum_acc = sum_acc + tsum
    m = max_acc - sum_acc * (1.0 / S)
    m_ref[...] = m.reshape(1, 1, L)


def _topk_kernel(m_ref, mtop_ref, *, H, S, U):
    m = m_ref[...].reshape(H, S)
    col = jax.lax.broadcasted_iota(jnp.int32, (H, S), 1)
    lane = jax.lax.broadcasted_iota(jnp.int32, (H, 128), 1)
    idxv = jnp.full((H, 128), S + 1000, dtype=jnp.int32)

    def body(i, carry):
        m, idxv = carry
        cm = jnp.max(m, axis=1, keepdims=True)
        cand = jnp.where(m == cm, col, jnp.int32(2147480000))
        pos = jnp.min(cand, axis=1, keepdims=True)
        m = jnp.where(col == pos, -3e38, m)
        idxv = jnp.where(lane == i, pos, idxv)
        return m, idxv

    _, idxv = jax.lax.fori_loop(0, U, body, (m, idxv))
    mtop_ref[...] = idxv.reshape(H, 1, 128)


def _attn_ctx_kernel(q_ref, k_ref, v_ref, mtop_ref, idx_smem, out_ref, *,
                     L, S, D, U, scale):
    idxs = mtop_ref[0]
    idx64 = idxs[:, :64]
    idx_col = jnp.swapaxes(idx64, 0, 1)

    col = jax.lax.broadcasted_iota(jnp.int32, (64, S), 1)
    oh = (idx_col == col).astype(jnp.bfloat16)

    q = q_ref[...]
    k = k_ref[...]
    v = v_ref[...]

    qr = jax.lax.dot_general(
        oh, q.astype(jnp.bfloat16), (((1,), (0,)), ((), ())),
        preferred_element_type=jnp.float32)

    sc = jax.lax.dot_general(
        qr.astype(jnp.bfloat16), k.astype(jnp.bfloat16),
        (((1,), (1,)), ((), ())),
        preferred_element_type=jnp.float32) * scale

    sc = jnp.where(col > idx_col, -1e9, sc)
    sc = sc - jnp.max(sc, axis=1, keepdims=True)
    e = jnp.exp(sc)
    attn = e / jnp.sum(e, axis=1, keepdims=True)

    att = jax.lax.dot_general(
        attn.astype(jnp.bfloat16), v.astype(jnp.bfloat16),
        (((1,), (0,)), ((), ())),
        preferred_element_type=jnp.float32)

    ctx = v
    sh = 1
    while sh < L:
        ctx = ctx + jnp.concatenate(
            [jnp.zeros((sh, D), jnp.float32), ctx[:L - sh, :]], axis=0)
        sh *= 2
    out_ref[...] = ctx

    for i in range(U):
        row = idx_smem[0, 0, i]
        out_ref[pl.ds(row, 1), :] = att[i:i + 1, :]


def _build_cmat(L, S, u):
    idx_key = jax.random.key(42)
    index_sample = jax.random.randint(idx_key, (L, u), 0, S)
    cnt = jnp.zeros((L, S), jnp.float32).at[
        jnp.arange(L)[:, None], index_sample].add(1.0)
    gate = jnp.where(cnt > 0.0, 0.0, -3e38).astype(jnp.float32)
    return cnt, gate


@functools.lru_cache(maxsize=None)
def _cmat_eager(L, S, u):
    with jax.ensure_compile_time_eval():
        return jax.block_until_ready(_build_cmat(L, S, u))


def _sample_consts(L, S, u):
    try:
        return _cmat_eager(L, S, u)
    except Exception:
        return _build_cmat(L, S, u)


def kernel(queries, keys, values):
    B, L, H, D = queries.shape
    S = keys.shape[1]
    u = min(max(1, int(_FACTOR * np.log(max(L, 2)))), L)
    cnt, gate = _sample_consts(L, S, u)
    return _run(queries, keys, values, cnt, gate)


@jax.jit
def _run(queries, keys, values, cmat, cgate):
    B, L, H, D = queries.shape
    S = keys.shape[1]
    U = min(max(1, int(_FACTOR * np.log(max(S, 2)))), S)
    scale = 1.0 / np.sqrt(D)

    Qs = queries.reshape(L, H * D)
    Ks = keys.reshape(L, H * D)
    Vs = values.reshape(L, H * D)

    m_call = pl.pallas_call(
        functools.partial(_m_kernel, L=L, S=S, D=D, KT=512),
        grid=(H,),
        in_specs=[
            pl.BlockSpec((L, D), lambda h: (0, h)),
            pl.BlockSpec((S, D), lambda h: (0, h)),
            pl.BlockSpec((L, S), lambda h: (0, 0)),
            pl.BlockSpec((L, S), lambda h: (0, 0)),
        ],
        out_specs=pl.BlockSpec((1, 1, L), lambda h: (h, 0, 0)),
        out_shape=jax.ShapeDtypeStruct((H, 1, L), jnp.float32),
    )
    m = m_call(Qs, Ks, cmat, cgate)

    topk_call = pl.pallas_call(
        functools.partial(_topk_kernel, H=H, S=S, U=U),
        grid=(1,),
        in_specs=[pl.BlockSpec((H, 1, S), lambda i: (0, 0, 0))],
        out_specs=pl.BlockSpec((H, 1, 128), lambda i: (0, 0, 0)),
        out_shape=jax.ShapeDtypeStruct((H, 1, 128), jnp.int32),
    )
    mtop = topk_call(m)

    attn_ctx = pl.pallas_call(
        functools.partial(_attn_ctx_kernel, L=L, S=S, D=D, U=U, scale=scale),
        grid=(H,),
        in_specs=[
            pl.BlockSpec((L, D), lambda h: (0, h)),
            pl.BlockSpec((S, D), lambda h: (0, h)),
            pl.BlockSpec((S, D), lambda h: (0, h)),
            pl.BlockSpec((1, 1, 128), lambda h: (h, 0, 0)),
            pl.BlockSpec((1, 1, 128), lambda h: (h, 0, 0),
                         memory_space=pltpu.SMEM),
        ],
        out_specs=pl.BlockSpec((S, D), lambda h: (0, h)),
        out_shape=jax.ShapeDtypeStruct((S, H * D), jnp.float32),
    )
    out = attn_ctx(Qs, Ks, Vs, mtop, mtop)
    return out.reshape(B, L, H, D)
